# speedup vs baseline: 7.9865x; 7.9865x over previous
import functools

import jax
import jax.numpy as jnp
from jax import lax
from jax.experimental import pallas as pl
from jax.experimental.pallas import tpu as pltpu
from jax.experimental.pallas import tpu_sc as plsc

_VOCAB = 1000000
_EMBED = 16
_BATCH = 16384
_HIST = 200

_NC, _NS, _L = 2, 16, 16
_NW = _NC * _NS
_WPT = _BATCH // _NW
_CB = _WPT // 128
_JC = 4
_NJ = _HIST // _JC


_ROWS = _VOCAB * _EMBED // 128
_RB = 5000


def _tv_body(t_ref, wd_ref, b_ref, o_ref):
    o_ref[...] = (
        jnp.dot(t_ref[...], wd_ref[...], preferred_element_type=jnp.float32)
        + b_ref[0, 0]
    )


def _compute_tv(table, W, b):
    t2 = table.reshape(_ROWS, 128)
    wd = jnp.kron(jnp.eye(8, dtype=jnp.float32), W.reshape(_EMBED, 1))
    tv2 = pl.pallas_call(
        _tv_body,
        grid=(_ROWS // _RB,),
        in_specs=[
            pl.BlockSpec((_RB, 128), lambda i: (i, 0)),
            pl.BlockSpec((128, 8), lambda i: (0, 0)),
            pl.BlockSpec(memory_space=pltpu.SMEM),
        ],
        out_specs=pl.BlockSpec((_RB, 8), lambda i: (i, 0)),
        out_shape=jax.ShapeDtypeStruct((_ROWS, 8), jnp.float32),
    )(t2, wd, b.reshape(1, 1))
    return tv2.reshape(_VOCAB)


def _sc_pool(xt_hbm, tv_hbm, o_hbm, idx_v, val_v, acc_v, sem):
    wid = lax.axis_index("s") * _NC + lax.axis_index("c")

    zero = jnp.zeros((_L,), jnp.float32)
    for c in range(_CB):
        for k in range(128 // _L):
            acc_v[c, pl.ds(k * _L, _L)] = zero

    @pl.loop(0, _NJ)
    def _(jc):
        pltpu.sync_copy(xt_hbm.at[pl.ds(jc * _JC, _JC), wid], idx_v)
        for j in range(_JC):
            for c in range(_CB):
                pltpu.make_async_copy(
                    tv_hbm.at[idx_v.at[j, c]], val_v.at[j, c], sem
                ).start()
        for j in range(_JC):
            for c in range(_CB):
                pltpu.make_async_copy(
                    tv_hbm.at[idx_v.at[j, c]], val_v.at[j, c], sem
                ).wait()
        for c in range(_CB):
            for k in range(128 // _L):
                sl = pl.ds(k * _L, _L)
                s = acc_v[c, sl]
                for j in range(_JC):
                    s = s + val_v[j, c, sl]
                acc_v[c, sl] = s

    inv = jnp.float32(1.0 / _HIST)
    one = jnp.float32(1.0)
    for c in range(_CB):
        for k in range(128 // _L):
            sl = pl.ds(k * _L, _L)
            z = acc_v[c, sl] * inv
            acc_v[c, sl] = one / (one + jnp.exp(-z))
    pltpu.sync_copy(acc_v, o_hbm.at[wid])


def _pooled_probs(xt4, tv):
    sc = pl.kernel(
        _sc_pool,
        out_type=jax.ShapeDtypeStruct((_NW, _CB, 128), jnp.float32),
        mesh=plsc.VectorSubcoreMesh(core_axis_name="c", subcore_axis_name="s"),
        scratch_types=[
            pltpu.VMEM((_JC, _CB, 128), jnp.int32),
            pltpu.VMEM((_JC, _CB, 128), jnp.float32),
            pltpu.VMEM((_CB, 128), jnp.float32),
            pltpu.SemaphoreType.DMA,
        ],
    )
    return sc(xt4, tv)


def kernel(x, table, W, b):
    tv = _compute_tv(table, W, b)
    xt4 = x.T.reshape(_HIST, _NW, _CB, 128)
    probs = _pooled_probs(xt4, tv)
    return probs.reshape(_BATCH, 1)


# speedup vs baseline: 9.0046x; 1.1275x over previous
import functools

import jax
import jax.numpy as jnp
from jax import lax
from jax.experimental import pallas as pl
from jax.experimental.pallas import tpu as pltpu
from jax.experimental.pallas import tpu_sc as plsc

_VOCAB = 1000000
_EMBED = 16
_BATCH = 16384
_HIST = 200

_NC, _NS, _L = 2, 16, 16
_NW = _NC * _NS
_WPT = _BATCH // _NW
_CB = _WPT // 128
_JC = 4
_NJ = _HIST // _JC


_RT = 16384
_TGRID = (_VOCAB + _RT - 1) // _RT
_VPAD = _TGRID * _RT


def _tv_body(t_ref, wd_ref, b_ref, o_ref):
    t = t_ref[...]
    s = jnp.sum(t * wd_ref[...], axis=1) + b_ref[0, 0]
    o_ref[...] = s.reshape(_RT // 128, 128)


def _compute_tv(table, W, b):
    tv2 = pl.pallas_call(
        _tv_body,
        grid=(_TGRID,),
        in_specs=[
            pl.BlockSpec((_RT, _EMBED), lambda i: (i, 0)),
            pl.BlockSpec((1, _EMBED), lambda i: (0, 0)),
            pl.BlockSpec(memory_space=pltpu.SMEM),
        ],
        out_specs=pl.BlockSpec((_RT // 128, 128), lambda i: (i, 0)),
        out_shape=jax.ShapeDtypeStruct((_VPAD // 128, 128), jnp.float32),
    )(table, W, b.reshape(1, 1))
    return tv2.reshape(_VPAD)


def _sc_pool(xt_hbm, tv_hbm, o_hbm, idx_v, val_v, acc_v, sem):
    wid = lax.axis_index("s") * _NC + lax.axis_index("c")

    zero = jnp.zeros((_L,), jnp.float32)
    for c in range(_CB):
        for k in range(128 // _L):
            acc_v[c, pl.ds(k * _L, _L)] = zero

    @pl.loop(0, _NJ)
    def _(jc):
        pltpu.sync_copy(xt_hbm.at[pl.ds(jc * _JC, _JC), wid], idx_v)
        for j in range(_JC):
            for c in range(_CB):
                pltpu.make_async_copy(
                    tv_hbm.at[idx_v.at[j, c]], val_v.at[j, c], sem
                ).start()
        for j in range(_JC):
            for c in range(_CB):
                pltpu.make_async_copy(
                    tv_hbm.at[idx_v.at[j, c]], val_v.at[j, c], sem
                ).wait()
        for c in range(_CB):
            for k in range(128 // _L):
                sl = pl.ds(k * _L, _L)
                s = acc_v[c, sl]
                for j in range(_JC):
                    s = s + val_v[j, c, sl]
                acc_v[c, sl] = s

    inv = jnp.float32(1.0 / _HIST)
    one = jnp.float32(1.0)
    for c in range(_CB):
        for k in range(128 // _L):
            sl = pl.ds(k * _L, _L)
            z = acc_v[c, sl] * inv
            acc_v[c, sl] = one / (one + jnp.exp(-z))
    pltpu.sync_copy(acc_v, o_hbm.at[wid])


def _pooled_probs(xt4, tv):
    sc = pl.kernel(
        _sc_pool,
        out_type=jax.ShapeDtypeStruct((_NW, _CB, 128), jnp.float32),
        mesh=plsc.VectorSubcoreMesh(core_axis_name="c", subcore_axis_name="s"),
        scratch_types=[
            pltpu.VMEM((_JC, _CB, 128), jnp.int32),
            pltpu.VMEM((_JC, _CB, 128), jnp.float32),
            pltpu.VMEM((_CB, 128), jnp.float32),
            pltpu.SemaphoreType.DMA,
        ],
    )
    return sc(xt4, tv)


def kernel(x, table, W, b):
    tv = _compute_tv(table, W, b)
    xt4 = x.T.reshape(_HIST, _NW, _CB, 128)
    probs = _pooled_probs(xt4, tv)
    return probs.reshape(_BATCH, 1)


# speedup vs baseline: 9.2945x; 1.0322x over previous
import dataclasses
import functools

import jax
import jax.numpy as jnp
from jax import lax
from jax.experimental import pallas as pl
from jax.experimental.pallas import tpu as pltpu
from jax.experimental.pallas import tpu_sc as plsc

_VOCAB = 1000000
_EMBED = 16
_BATCH = 16384
_HIST = 200

_NC, _NS, _L = 2, 16, 16
_NW = _NC * _NS
_WPT = _BATCH // _NW
_CB = _WPT // 128
_JC = 4
_NJ = _HIST // _JC


_RT = 16384
_TGRID = (_VOCAB + _RT - 1) // _RT
_VPAD = _TGRID * _RT


def _tv_body(t_ref, wd_ref, b_ref, o_ref):
    t = t_ref[...]
    s = jnp.sum(t * wd_ref[...], axis=1) + b_ref[0, 0]
    o_ref[...] = s.reshape(_RT // 128, 128)


def _compute_tv(table, W, b):
    tv2 = pl.pallas_call(
        _tv_body,
        grid=(_TGRID,),
        in_specs=[
            pl.BlockSpec((_RT, _EMBED), lambda i: (i, 0)),
            pl.BlockSpec((1, _EMBED), lambda i: (0, 0)),
            pl.BlockSpec(memory_space=pltpu.SMEM),
        ],
        out_specs=pl.BlockSpec((_RT // 128, 128), lambda i: (i, 0)),
        out_shape=jax.ShapeDtypeStruct((_VPAD // 128, 128), jnp.float32),
    )(table, W, b.reshape(1, 1))
    return tv2.reshape(_VPAD)


_RC = 16
_NCH = _WPT // _RC
_VW = 208


def _sc_pool(x_hbm, tv_hbm, o_hbm, idx_v, val_v, srow_v, osum_v, sem):
    wid = lax.axis_index("s") * _NC + lax.axis_index("c")
    base = wid * _WPT

    zero = jnp.zeros((_L,), jnp.float32)
    for r in range(_RC):
        val_v[r, pl.ds(_VW - _L, _L)] = zero

    iota16 = lax.iota(jnp.int32, _L) * _L

    @pl.loop(0, _NCH)
    def _(ch):
        pltpu.sync_copy(x_hbm.at[pl.ds(base + ch * _RC, _RC), :], idx_v)
        for r in range(_RC):
            pltpu.make_async_copy(
                tv_hbm.at[idx_v.at[r, pl.ds(0, 128)]],
                val_v.at[r, pl.ds(0, 128)], sem,
            ).start()
            pltpu.make_async_copy(
                tv_hbm.at[idx_v.at[r, pl.ds(128, _HIST - 128)]],
                val_v.at[r, pl.ds(128, _HIST - 128)], sem,
            ).start()
        for r in range(_RC):
            pltpu.make_async_copy(
                tv_hbm.at[idx_v.at[r, pl.ds(0, 128)]],
                val_v.at[r, pl.ds(0, 128)], sem,
            ).wait()
            pltpu.make_async_copy(
                tv_hbm.at[idx_v.at[r, pl.ds(128, _HIST - 128)]],
                val_v.at[r, pl.ds(128, _HIST - 128)], sem,
            ).wait()
        for r in range(_RC):
            s = val_v[r, pl.ds(0, _L)]
            for k in range(1, _VW // _L):
                s = s + val_v[r, pl.ds(k * _L, _L)]
            srow_v[pl.ds(r * _L, _L)] = s
        y = plsc.load_gather(srow_v, [iota16])
        for k in range(1, _L):
            y = y + plsc.load_gather(srow_v, [iota16 + k])
        osum_v[pl.ds(ch * _RC, _RC)] = y

    inv = jnp.float32(1.0 / _HIST)
    one = jnp.float32(1.0)
    for k in range(_WPT // _L):
        sl = pl.ds(k * _L, _L)
        z = osum_v[sl] * inv
        osum_v[sl] = one / (one + jnp.exp(-z))
    pltpu.sync_copy(osum_v, o_hbm.at[wid])


def _sc_compiler_params():
    cp = pltpu.CompilerParams()
    if "needs_layout_passes" in pltpu.CompilerParams.__dataclass_fields__:
        cp = dataclasses.replace(cp, needs_layout_passes=False)
    return cp


def _pooled_probs(x, tv):
    sc = pl.kernel(
        _sc_pool,
        out_type=jax.ShapeDtypeStruct((_NW, _WPT), jnp.float32),
        mesh=plsc.VectorSubcoreMesh(core_axis_name="c", subcore_axis_name="s"),
        compiler_params=_sc_compiler_params(),
        scratch_types=[
            pltpu.VMEM((_RC, _HIST), jnp.int32),
            pltpu.VMEM((_RC, _VW), jnp.float32),
            pltpu.VMEM((_RC * _L,), jnp.float32),
            pltpu.VMEM((_WPT,), jnp.float32),
            pltpu.SemaphoreType.DMA,
        ],
    )
    return sc(x, tv)


def kernel(x, table, W, b):
    tv = _compute_tv(table, W, b)
    probs = _pooled_probs(x, tv)
    return probs.reshape(_BATCH, 1)
